# Initial kernel scaffold; baseline (speedup 1.0000x reference)
import jax
import jax.numpy as jnp
from jax.experimental import pallas as pl


def kernel(x, prob, W_qkv, W_out, b_out):
    raise NotImplementedError("write your pallas kernel here")


# speedup vs baseline: 6.9304x; 6.9304x over previous
import functools
import numpy as np
import jax
import jax.numpy as jnp
from jax import lax
from jax.experimental import pallas as pl
from jax.experimental.pallas import tpu as pltpu

DIM = 256
HEADS = 8
DIM_HEAD = 32
WIN = 16
IMG = 128
B = 4
C_PROB = 21
HP = 64
NG = 15
KEEP = 44
SCALE = DIM_HEAD ** -0.5
NEG = -1e30


_BM = np.zeros((16, HP), np.float32)
for _r in range(NG):
    _BM[_r, 4 * _r:4 * _r + 8] = 1.0

_POS = (np.arange(16, dtype=np.float64) + 0.5) * 15.0 / 16.0
_LO = np.floor(_POS).astype(np.int32)
_FR = (_POS - _LO).astype(np.float32)

_R8 = np.zeros((IMG, 16), np.float32)
_R8[np.arange(IMG), np.arange(IMG) // 8] = 1.0


def _resample(patch, fr):
    fy = fr.reshape(16)[:, None, None]
    fx = fr.reshape(16)[None, :, None]
    a = jnp.concatenate([patch[0:8], patch[7:15]], axis=0)
    bq = jnp.concatenate([patch[1:9], patch[8:16]], axis=0)
    t = a * (1.0 - fy) + bq * fy
    a = jnp.concatenate([t[:, 0:8], t[:, 7:15]], axis=1)
    bq = jnp.concatenate([t[:, 1:9], t[:, 8:16]], axis=1)
    return a * (1.0 - fx) + bq * fx


def _nms_body(bm_ref, prob_ref, sel_ref):
    p = prob_ref[0]
    ent = -jnp.sum(p * jnp.log2(p + 1e-10), axis=0)
    bm = bm_ref[...]
    s = jnp.dot(bm, jnp.dot(ent, bm.T, preferred_element_type=jnp.float32),
                preferred_element_type=jnp.float32) * (1.0 / 64.0)
    iy = lax.broadcasted_iota(jnp.int32, (16, 16), 0)
    ix = lax.broadcasted_iota(jnp.int32, (16, 16), 1)
    valid = (iy < NG) & (ix < NG)
    scores = jnp.where(valid, s, NEG)
    lin = iy * 16 + ix
    lane = lax.broadcasted_iota(jnp.int32, (1, 64), 1)

    def body(k, carry):
        sc, sel = carry
        m = jnp.max(sc)
        widx = jnp.min(jnp.where(sc == m, lin, 1 << 20))
        wy = widx // 16
        wx = widx - wy * 16
        wid = wy * NG + wx
        sel = jnp.where(lane == k, wid, sel)
        manh = jnp.abs(iy - wy) + jnp.abs(ix - wx)
        sc = jnp.where(manh <= 1, NEG, sc)
        return sc, sel

    _, sel = lax.fori_loop(0, KEEP, body, (scores, jnp.zeros((1, 64), jnp.int32)))
    sel_ref[0] = sel


def _run_nms(prob):
    return pl.pallas_call(
        _nms_body,
        grid=(B,),
        in_specs=[pl.BlockSpec((16, HP), lambda i: (0, 0)),
                  pl.BlockSpec((1, C_PROB, HP, HP), lambda i: (i, 0, 0, 0))],
        out_specs=pl.BlockSpec((1, 1, 64), lambda i: (i, 0, 0)),
        out_shape=jax.ShapeDtypeStruct((B, 1, 64), jnp.int32),
    )(jnp.asarray(_BM), prob)


def _attn_body(wy_ref, wx_ref, x_ref, wqkv_ref, wout_ref, bout_ref, fr_ref,
               r8_ref, out_ref, pickt_ref):
    i = pl.program_id(0)
    k = pl.program_id(1)
    wy = wy_ref[i, k]
    wx = wx_ref[i, k]
    sy = wy * 8
    sx = wx * 8

    @pl.when(k == 0)
    def _():
        out_ref[...] = jnp.zeros_like(out_ref)
        pickt_ref[...] = jnp.zeros_like(pickt_ref)

    patch = x_ref[0, pl.ds(sy, WIN), pl.ds(sx, WIN), :]
    tokens = _resample(patch, fr_ref[...]).reshape(WIN * WIN, DIM)

    qkv = jnp.dot(tokens, wqkv_ref[...], preferred_element_type=jnp.float32)
    outs = []
    for h in range(HEADS):
        q = qkv[:, h * DIM_HEAD:(h + 1) * DIM_HEAD]
        kk = qkv[:, 256 + h * DIM_HEAD:256 + (h + 1) * DIM_HEAD]
        v = qkv[:, 512 + h * DIM_HEAD:512 + (h + 1) * DIM_HEAD]
        dots = lax.dot_general(q, kk, (((1,), (1,)), ((), ())),
                               preferred_element_type=jnp.float32) * SCALE
        dots = dots - jnp.max(dots, axis=1, keepdims=True)
        e = jnp.exp(dots)
        attn = e / jnp.sum(e, axis=1, keepdims=True)
        outs.append(jnp.dot(attn, v, preferred_element_type=jnp.float32))
    out_t = jnp.concatenate(outs, axis=1)
    y = jnp.dot(out_t, wout_ref[...], preferred_element_type=jnp.float32)
    y = y + bout_ref[...]
    ypatch = y.reshape(WIN, WIN, DIM)

    out_ref[0, pl.ds(sy, WIN), pl.ds(sx, WIN), :] += ypatch
    iy = lax.broadcasted_iota(jnp.int32, (16, 16), 0)
    ix = lax.broadcasted_iota(jnp.int32, (16, 16), 1)
    pickt_ref[...] += ((iy == wx) & (ix == wy)).astype(jnp.float32)

    @pl.when(k == KEEP - 1)
    def _():
        pt = pickt_ref[...]
        z_r = jnp.zeros((1, 16), jnp.float32)
        z_c = jnp.zeros((16, 1), jnp.float32)
        ct = pt + jnp.concatenate([z_r, pt[:15]], axis=0)
        ct = ct + jnp.concatenate([z_c, ct[:, :15]], axis=1)
        rec_t = jnp.dot(r8_ref[...], 1.0 / (ct + 1e-10),
                        preferred_element_type=jnp.float32)
        for cy in range(16):
            scale = lax.broadcast_in_dim(
                rec_t[:, cy:cy + 1], (8, IMG, DIM), (1, 2))
            sl = pl.ds(8 * cy, 8)
            out_ref[0, sl] = x_ref[0, sl] + out_ref[0, sl] * scale


def _run_attn(x4d, wy, wx, W_qkv, W_out, b_out):
    grid_spec = pltpu.PrefetchScalarGridSpec(
        num_scalar_prefetch=2,
        grid=(B, KEEP),
        in_specs=[
            pl.BlockSpec((1, IMG, IMG, DIM), lambda i, k, *_: (i, 0, 0, 0),
                         pipeline_mode=pl.Buffered(buffer_count=1)),
            pl.BlockSpec((DIM, 3 * DIM), lambda i, k, *_: (0, 0)),
            pl.BlockSpec((DIM, DIM), lambda i, k, *_: (0, 0)),
            pl.BlockSpec((1, DIM), lambda i, k, *_: (0, 0)),
            pl.BlockSpec((1, 16), lambda i, k, *_: (0, 0)),
            pl.BlockSpec((IMG, 16), lambda i, k, *_: (0, 0)),
        ],
        out_specs=pl.BlockSpec((1, IMG, IMG, DIM), lambda i, k, *_: (i, 0, 0, 0),
                               pipeline_mode=pl.Buffered(buffer_count=1)),
        scratch_shapes=[pltpu.VMEM((16, 16), jnp.float32)],
    )
    return pl.pallas_call(
        _attn_body,
        grid_spec=grid_spec,
        out_shape=jax.ShapeDtypeStruct((B, IMG, IMG, DIM), jnp.float32),
        compiler_params=pltpu.CompilerParams(
            dimension_semantics=("arbitrary", "arbitrary")),
    )(wy, wx, x4d, W_qkv, W_out, b_out, jnp.asarray(_FR).reshape(1, 16),
      jnp.asarray(_R8))


@jax.jit
def kernel(x, prob, W_qkv, W_out, b_out):
    sel = _run_nms(prob)[:, 0, :KEEP]
    wy = (sel // NG).astype(jnp.int32)
    wx = (sel - (sel // NG) * NG).astype(jnp.int32)
    x4d = x.reshape(B, IMG, IMG, DIM)
    out = _run_attn(x4d, wy, wx, W_qkv, W_out, b_out.reshape(1, DIM))
    return out.reshape(B, IMG * IMG, DIM)


# speedup vs baseline: 7.0184x; 1.0127x over previous
import functools
import numpy as np
import jax
import jax.numpy as jnp
from jax import lax
from jax.experimental import pallas as pl
from jax.experimental.pallas import tpu as pltpu

DIM = 256
HEADS = 8
DIM_HEAD = 32
WIN = 16
IMG = 128
B = 4
C_PROB = 21
HP = 64
NG = 15
KEEP = 44
SCALE = DIM_HEAD ** -0.5
NEG = -1e30


_BM = np.zeros((16, HP), np.float32)
for _r in range(NG):
    _BM[_r, 4 * _r:4 * _r + 8] = 1.0

_POS = (np.arange(16, dtype=np.float64) + 0.5) * 15.0 / 16.0
_LO = np.floor(_POS).astype(np.int32)
_FR = (_POS - _LO).astype(np.float32)

_R8 = np.zeros((IMG, 16), np.float32)
_R8[np.arange(IMG), np.arange(IMG) // 8] = 1.0


def _resample(patch, fr):
    fy = fr.reshape(16)[:, None, None]
    fx = fr.reshape(16)[None, :, None]
    a = jnp.concatenate([patch[0:8], patch[7:15]], axis=0)
    bq = jnp.concatenate([patch[1:9], patch[8:16]], axis=0)
    t = a * (1.0 - fy) + bq * fy
    a = jnp.concatenate([t[:, 0:8], t[:, 7:15]], axis=1)
    bq = jnp.concatenate([t[:, 1:9], t[:, 8:16]], axis=1)
    return a * (1.0 - fx) + bq * fx


def _nms_body(bm_ref, prob_ref, sel_ref):
    p = prob_ref[0]
    ent = -jnp.sum(p * jnp.log2(p + 1e-10), axis=0)
    bm = bm_ref[...]
    s = jnp.dot(bm, jnp.dot(ent, bm.T, preferred_element_type=jnp.float32),
                preferred_element_type=jnp.float32) * (1.0 / 64.0)
    iy = lax.broadcasted_iota(jnp.int32, (16, 16), 0)
    ix = lax.broadcasted_iota(jnp.int32, (16, 16), 1)
    valid = (iy < NG) & (ix < NG)
    scores = jnp.where(valid, s, NEG)
    lin = iy * 16 + ix
    lane = lax.broadcasted_iota(jnp.int32, (1, 64), 1)

    def body(k, carry):
        sc, sel = carry
        m = jnp.max(sc)
        widx = jnp.min(jnp.where(sc == m, lin, 1 << 20))
        wy = widx // 16
        wx = widx - wy * 16
        wid = wy * NG + wx
        sel = jnp.where(lane == k, wid, sel)
        manh = jnp.abs(iy - wy) + jnp.abs(ix - wx)
        sc = jnp.where(manh <= 1, NEG, sc)
        return sc, sel

    _, sel = lax.fori_loop(0, KEEP, body, (scores, jnp.zeros((1, 64), jnp.int32)))
    sel_ref[0] = sel


def _run_nms(prob):
    return pl.pallas_call(
        _nms_body,
        grid=(B,),
        in_specs=[pl.BlockSpec((16, HP), lambda i: (0, 0)),
                  pl.BlockSpec((1, C_PROB, HP, HP), lambda i: (i, 0, 0, 0))],
        out_specs=pl.BlockSpec((1, 1, 64), lambda i: (i, 0, 0)),
        out_shape=jax.ShapeDtypeStruct((B, 1, 64), jnp.int32),
    )(jnp.asarray(_BM), prob)


def _attn_body(wy_ref, wx_ref, x_ref, wqkv_ref, wout_ref, bout_ref, fr_ref,
               r8_ref, out_ref, pickt_ref):
    i = pl.program_id(0)
    k = pl.program_id(1)
    wy = wy_ref[i, k]
    wx = wx_ref[i, k]
    sy = wy * 8
    sx = wx * 8

    @pl.when(k == 0)
    def _():
        out_ref[...] = jnp.zeros_like(out_ref)
        pickt_ref[...] = jnp.zeros_like(pickt_ref)

    patch = x_ref[0, pl.ds(sy, WIN), pl.ds(sx, WIN), :]
    tokens = _resample(patch, fr_ref[...]).reshape(WIN * WIN, DIM)

    bf = jnp.bfloat16
    qkv = jnp.dot(tokens.astype(bf), wqkv_ref[...],
                  preferred_element_type=jnp.float32)
    qkv_bf = qkv.astype(bf)
    outs = []
    for h in range(HEADS):
        q = qkv_bf[:, h * DIM_HEAD:(h + 1) * DIM_HEAD]
        kk = qkv_bf[:, 256 + h * DIM_HEAD:256 + (h + 1) * DIM_HEAD]
        v = qkv_bf[:, 512 + h * DIM_HEAD:512 + (h + 1) * DIM_HEAD]
        dots = lax.dot_general(q, kk, (((1,), (1,)), ((), ())),
                               preferred_element_type=jnp.float32) * SCALE
        dots = dots - jnp.max(dots, axis=1, keepdims=True)
        e = jnp.exp(dots)
        attn = (e / jnp.sum(e, axis=1, keepdims=True)).astype(bf)
        outs.append(jnp.dot(attn, v, preferred_element_type=jnp.float32))
    out_t = jnp.concatenate(outs, axis=1).astype(bf)
    y = jnp.dot(out_t, wout_ref[...], preferred_element_type=jnp.float32)
    y = y + bout_ref[...]
    ypatch = y.reshape(WIN, WIN, DIM)

    out_ref[0, pl.ds(sy, WIN), pl.ds(sx, WIN), :] += ypatch
    iy = lax.broadcasted_iota(jnp.int32, (16, 16), 0)
    ix = lax.broadcasted_iota(jnp.int32, (16, 16), 1)
    pickt_ref[...] += ((iy == wx) & (ix == wy)).astype(jnp.float32)

    @pl.when(k == KEEP - 1)
    def _():
        pt = pickt_ref[...]
        z_r = jnp.zeros((1, 16), jnp.float32)
        z_c = jnp.zeros((16, 1), jnp.float32)
        ct = pt + jnp.concatenate([z_r, pt[:15]], axis=0)
        ct = ct + jnp.concatenate([z_c, ct[:, :15]], axis=1)
        rec_t = jnp.dot(r8_ref[...], 1.0 / (ct + 1e-10),
                        preferred_element_type=jnp.float32)
        for cy in range(16):
            scale = lax.broadcast_in_dim(
                rec_t[:, cy:cy + 1], (8, IMG, DIM), (1, 2))
            sl = pl.ds(8 * cy, 8)
            out_ref[0, sl] = x_ref[0, sl] + out_ref[0, sl] * scale


def _run_attn(x4d, wy, wx, W_qkv, W_out, b_out):
    grid_spec = pltpu.PrefetchScalarGridSpec(
        num_scalar_prefetch=2,
        grid=(B, KEEP),
        in_specs=[
            pl.BlockSpec((1, IMG, IMG, DIM), lambda i, k, *_: (i, 0, 0, 0),
                         pipeline_mode=pl.Buffered(buffer_count=1)),
            pl.BlockSpec((DIM, 3 * DIM), lambda i, k, *_: (0, 0)),
            pl.BlockSpec((DIM, DIM), lambda i, k, *_: (0, 0)),
            pl.BlockSpec((1, DIM), lambda i, k, *_: (0, 0)),
            pl.BlockSpec((1, 16), lambda i, k, *_: (0, 0)),
            pl.BlockSpec((IMG, 16), lambda i, k, *_: (0, 0)),
        ],
        out_specs=pl.BlockSpec((1, IMG, IMG, DIM), lambda i, k, *_: (i, 0, 0, 0),
                               pipeline_mode=pl.Buffered(buffer_count=1)),
        scratch_shapes=[pltpu.VMEM((16, 16), jnp.float32)],
    )
    return pl.pallas_call(
        _attn_body,
        grid_spec=grid_spec,
        out_shape=jax.ShapeDtypeStruct((B, IMG, IMG, DIM), jnp.float32),
        compiler_params=pltpu.CompilerParams(
            dimension_semantics=("arbitrary", "arbitrary")),
    )(wy, wx, x4d, W_qkv.astype(jnp.bfloat16), W_out.astype(jnp.bfloat16),
      b_out, jnp.asarray(_FR).reshape(1, 16), jnp.asarray(_R8))


@jax.jit
def kernel(x, prob, W_qkv, W_out, b_out):
    sel = _run_nms(prob)[:, 0, :KEEP]
    wy = (sel // NG).astype(jnp.int32)
    wx = (sel - (sel // NG) * NG).astype(jnp.int32)
    x4d = x.reshape(B, IMG, IMG, DIM)
    out = _run_attn(x4d, wy, wx, W_qkv, W_out, b_out.reshape(1, DIM))
    return out.reshape(B, IMG * IMG, DIM)
